# baseline (device time: 77773 ns/iter reference)
import contextlib
import os

import jax
import jax.numpy as jnp
from jax import lax
from jax.experimental import pallas as pl
from jax.experimental.pallas import tpu as pltpu

N_DEV = 4
N_CHUNK = 2
_SCOPES = os.environ.get("KERNEL_SCOPES", "0") == "1"


def _scope(name):
    return jax.named_scope(name) if _SCOPES else contextlib.nullcontext()


def kernel(x, w_mat):
    m_glob, k_shard = x.shape
    k_glob, n = w_mat.shape
    m_blk = m_glob // N_DEV
    k_blk = k_glob // N_DEV
    m_ch = m_blk // N_CHUNK

    def body(x_hbm, w_hbm, out_hbm, xstg, xb, comm, wstg, wb, acc,
             my_amax, amax_slots, xld_sems, wld_sems, send_sems, recv_sems,
             ax_send_sems, ax_recv_sems, out_sems):
        my = lax.axis_index("i")

        with _scope("barrier"):
            bsem = pltpu.get_barrier_semaphore()
            for k in range(1, N_DEV):
                pl.semaphore_signal(bsem, inc=1,
                                    device_id=((my + k) % N_DEV,),
                                    device_id_type=pl.DeviceIdType.MESH)
            pl.semaphore_wait(bsem, N_DEV - 1)

        def w_dma(src_off):
            e = (my + src_off) % N_DEV
            c = pltpu.make_async_copy(w_hbm.at[pl.ds(e * k_blk, k_blk), :],
                                      wstg.at[0], wld_sems.at[0])
            c.start()
            return c

        def x_dma(k, slot):
            tgt = (my + k) % N_DEV
            c = pltpu.make_async_copy(x_hbm.at[pl.ds(tgt * m_blk, m_blk), :],
                                      xstg.at[slot], xld_sems.at[slot])
            c.start()
            return c

        wd = w_dma(0)
        with _scope("xload_send"):
            order = [2, 1, 3, 0]
            dmas = {order[0]: x_dma(order[0], 0), order[1]: x_dma(order[1], 1)}
            hop = {}
            for i, k in enumerate(order):
                slot = i % 2
                dmas[k].wait()
                xb[k] = xstg[slot].astype(jnp.bfloat16)
                if k != 0:
                    for c in range(N_CHUNK):
                        r = pltpu.make_async_remote_copy(
                            src_ref=xb.at[k, pl.ds(c * m_ch, m_ch), :],
                            dst_ref=comm.at[N_DEV - k, pl.ds(c * m_ch, m_ch), :],
                            send_sem=send_sems.at[k - 1, c],
                            recv_sem=recv_sems.at[N_DEV - k, c],
                            device_id=((my + k) % N_DEV,),
                            device_id_type=pl.DeviceIdType.MESH,
                        )
                        r.start()
                        hop[(k, c)] = r
                if i + 2 < len(order):
                    dmas[order[i + 2]] = x_dma(order[i + 2], slot)

        w_chain = [3, 1, 2]

        def w_ready(j, wslot, next_i):
            nonlocal wd
            wd.wait()
            wb[wslot] = wstg[0].astype(jnp.bfloat16)
            if next_i < len(w_chain):
                wd = w_dma(w_chain[next_i])

        with _scope("gemm_local"):
            w_ready(0, 0, 0)
            acc[...] = jnp.dot(xb[0], wb[0],
                               preferred_element_type=jnp.float32)

        chunk_amax = []
        for i, k in enumerate((1, 3, 2)):
            j = N_DEV - k
            wslot = (i + 1) % 2
            with _scope(f"wprep#j={j}"):
                w_ready(j, wslot, i + 1)
            for c in range(N_CHUNK):
                with _scope(f"wait_recv#k={k}_c={c}"):
                    hop[(k, c)].wait()
                with _scope(f"gemm#k={k}_c={c}"):
                    rows = slice(c * m_ch, (c + 1) * m_ch)
                    acc[rows, :] += jnp.dot(comm[j, rows, :], wb[wslot],
                                            preferred_element_type=jnp.float32)
                if k == 2:
                    with _scope(f"amax#c={c}"):
                        chunk_amax.append(jnp.max(jnp.abs(acc[rows, :])))

        with _scope("amax_exchange"):
            local_amax = jnp.maximum(chunk_amax[0], chunk_amax[1])
            my_amax[...] = jnp.full((8, 128), local_amax, dtype=jnp.float32)
            ax = []
            for k in range(1, N_DEV):
                r = pltpu.make_async_remote_copy(
                    src_ref=my_amax,
                    dst_ref=amax_slots.at[N_DEV - k],
                    send_sem=ax_send_sems.at[k - 1],
                    recv_sem=ax_recv_sems.at[N_DEV - k],
                    device_id=((my + k) % N_DEV,),
                    device_id_type=pl.DeviceIdType.MESH,
                )
                r.start()
                ax.append(r)
            for r in ax:
                r.wait()
            g = jnp.maximum(local_amax, jnp.max(amax_slots[1:N_DEV]))

        with _scope("quant"):
            scale = g / 448.0
            ods = []
            for h in range(2):
                cols = slice(h * k_shard, (h + 1) * k_shard)
                q = (acc[:, cols] / scale).astype(jnp.float8_e4m3fn)
                xb[1 + h] = (q.astype(jnp.float32) * scale).astype(jnp.bfloat16)
                od = pltpu.make_async_copy(
                    xb.at[1 + h], out_hbm.at[:, pl.ds(h * k_shard, k_shard)],
                    out_sems.at[h])
                od.start()
                ods.append(od)
            for od in ods:
                od.wait()

    return pl.pallas_call(
        body,
        out_shape=jax.ShapeDtypeStruct((m_blk, n), jnp.bfloat16),
        in_specs=[
            pl.BlockSpec(memory_space=pl.ANY),
            pl.BlockSpec(memory_space=pl.ANY),
        ],
        out_specs=pl.BlockSpec(memory_space=pl.ANY),
        scratch_shapes=[
            pltpu.VMEM((2, m_blk, k_shard), jnp.float32),
            pltpu.VMEM((N_DEV, m_blk, k_shard), jnp.bfloat16),
            pltpu.VMEM((N_DEV, m_blk, k_shard), jnp.bfloat16),
            pltpu.VMEM((1, k_blk, n), jnp.float32),
            pltpu.VMEM((2, k_blk, n), jnp.bfloat16),
            pltpu.VMEM((m_blk, n), jnp.float32),
            pltpu.VMEM((8, 128), jnp.float32),
            pltpu.VMEM((N_DEV, 8, 128), jnp.float32),
            pltpu.SemaphoreType.DMA((2,)),
            pltpu.SemaphoreType.DMA((1,)),
            pltpu.SemaphoreType.DMA((3, N_CHUNK)),
            pltpu.SemaphoreType.DMA((N_DEV, N_CHUNK)),
            pltpu.SemaphoreType.DMA((3,)),
            pltpu.SemaphoreType.DMA((4,)),
            pltpu.SemaphoreType.DMA((2,)),
        ],
        compiler_params=pltpu.CompilerParams(
            collective_id=0,
            vmem_limit_bytes=63 * 1024 * 1024,
        ),
    )(x, w_mat)


# device time: 68046 ns/iter; 1.1429x vs baseline; 1.1429x over previous
import contextlib
import os

import jax
import jax.numpy as jnp
from jax import lax
from jax.experimental import pallas as pl
from jax.experimental.pallas import tpu as pltpu

N_DEV = 4
CHUNK_ROWS = {1: [512, 512], 3: [512, 512], 2: [256, 256, 256, 256]}
_OFFS = {k: [sum(v[:i]) for i in range(len(v))] for k, v in CHUNK_ROWS.items()}
_SCOPES = os.environ.get("KERNEL_SCOPES", "0") == "1"
_MODE = os.environ.get("KERNEL_MODE", "full")


def _scope(name):
    return jax.named_scope(name) if _SCOPES else contextlib.nullcontext()


def kernel(x, w_mat):
    m_glob, k_shard = x.shape
    k_glob, n = w_mat.shape
    m_blk = m_glob // N_DEV
    k_blk = k_glob // N_DEV

    def body(x_hbm, w_hbm, out_hbm, xstg, xb, comm, wstg, acc,
             my_amax, amax_slots, xld_sems, wld_sems, send_sems, recv_sems,
             ax_send_sems, ax_recv_sems, out_sems):
        my = lax.axis_index("i")

        with _scope("barrier_signal"):
            bsem = pltpu.get_barrier_semaphore()
            for k in range(1, N_DEV):
                pl.semaphore_signal(bsem, inc=1,
                                    device_id=((my + k) % N_DEV,),
                                    device_id_type=pl.DeviceIdType.MESH)
            pl.semaphore_wait(bsem, N_DEV - 1)

        def send_chunk(k, c):
            off, rows = _OFFS[k][c], CHUNK_ROWS[k][c]
            r = pltpu.make_async_remote_copy(
                src_ref=xb.at[k, pl.ds(off, rows), :],
                dst_ref=comm.at[N_DEV - k, pl.ds(off, rows), :],
                send_sem=send_sems.at[k - 1, c],
                recv_sem=recv_sems.at[N_DEV - k, c],
                device_id=((my + k) % N_DEV,),
                device_id_type=pl.DeviceIdType.MESH,
            )
            r.start()
            return r

        if _MODE.startswith("commbench"):
            ks = (1, 3) if _MODE == "commbench2" else (1, 3, 2)
            bhop = {}
            for k in ks:
                for c in range(len(CHUNK_ROWS[k])):
                    bhop[(k, c)] = send_chunk(k, c)
            for kc in bhop:
                bhop[kc].wait()
            return

        def w_dma(src_off, slot):
            e = (my + src_off) % N_DEV
            c = pltpu.make_async_copy(w_hbm.at[pl.ds(e * k_blk, k_blk), :],
                                      wstg.at[slot], wld_sems.at[slot])
            c.start()
            return c

        def x_dma(k, slot):
            tgt = (my + k) % N_DEV
            c = pltpu.make_async_copy(x_hbm.at[pl.ds(tgt * m_blk, m_blk), :],
                                      xstg.at[slot], xld_sems.at[slot])
            c.start()
            return c

        with _scope("xload_send"):
            order = [1, 3, 2, 0]
            dmas = {order[0]: x_dma(order[0], 0), order[1]: x_dma(order[1], 1)}
            hop = {}
            for i, k in enumerate(order):
                slot = i % 2
                dmas[k].wait()
                if k != 0:
                    for c, (off, rows) in enumerate(
                            zip(_OFFS[k], CHUNK_ROWS[k])):
                        xb[k, pl.ds(off, rows), :] = xstg[
                            slot, off:off + rows, :
                        ].astype(jnp.bfloat16)
                        hop[(k, c)] = send_chunk(k, c)
                if i + 2 < len(order):
                    dmas[order[i + 2]] = x_dma(order[i + 2], slot)
        local_slot = 1

        if _MODE == "commload":
            wd = {0: w_dma(0, 0), 3: w_dma(3, 1)}
            wd[0].wait()
            wd[1] = w_dma(1, 0)
            wd[3].wait()
            wd[2] = w_dma(2, 1)
            wd[1].wait()
            wd[2].wait()
            for kc in hop:
                hop[kc].wait()
            return

        wd = {0: w_dma(0, 0), 3: w_dma(3, 1)}

        with _scope("gemm_local"):
            wd[0].wait()
            acc[...] = jnp.dot(xstg[local_slot], wstg[0],
                               preferred_element_type=jnp.float32)
            wd[1] = w_dma(1, 0)

        chunk_amax = []
        for i, k in enumerate((1, 3, 2)):
            j = N_DEV - k
            wslot = (i + 1) % 2
            with _scope(f"wprep#j={j}"):
                wd[j].wait()
                if i == 1:
                    wd[2] = w_dma(2, 1)
            for c, (off, nrows) in enumerate(zip(_OFFS[k], CHUNK_ROWS[k])):
                with _scope(f"wait_recv#k={k}_c={c}"):
                    hop[(k, c)].wait()
                with _scope(f"gemm#k={k}_c={c}"):
                    rows = slice(off, off + nrows)
                    acc[rows, :] += jnp.dot(comm[j, rows, :], wstg[wslot],
                                            preferred_element_type=jnp.float32)
                if k == 2:
                    with _scope(f"amax#c={c}"):
                        chunk_amax.append(jnp.max(jnp.abs(acc[rows, :])))

        with _scope("amax_exchange"):
            local_amax = chunk_amax[0]
            for cm in chunk_amax[1:]:
                local_amax = jnp.maximum(local_amax, cm)
            my_amax[...] = jnp.full((8, 128), local_amax, dtype=jnp.float32)
            ax = []
            for k in range(1, N_DEV):
                r = pltpu.make_async_remote_copy(
                    src_ref=my_amax,
                    dst_ref=amax_slots.at[N_DEV - k],
                    send_sem=ax_send_sems.at[k - 1],
                    recv_sem=ax_recv_sems.at[N_DEV - k],
                    device_id=((my + k) % N_DEV,),
                    device_id_type=pl.DeviceIdType.MESH,
                )
                r.start()
                ax.append(r)
            for r in ax:
                r.wait()
            g = jnp.maximum(local_amax, jnp.max(amax_slots[1:N_DEV]))

        with _scope("quant"):
            scale = g / 448.0
            ods = []
            for h in range(2):
                cols = slice(h * k_shard, (h + 1) * k_shard)
                q = (acc[:, cols] / scale).astype(jnp.float8_e4m3fn)
                xb[1 + h] = (q.astype(jnp.float32) * scale).astype(jnp.bfloat16)
                od = pltpu.make_async_copy(
                    xb.at[1 + h], out_hbm.at[:, pl.ds(h * k_shard, k_shard)],
                    out_sems.at[h])
                od.start()
                ods.append(od)
            for od in ods:
                od.wait()

    return pl.pallas_call(
        body,
        out_shape=jax.ShapeDtypeStruct((m_blk, n), jnp.bfloat16),
        in_specs=[
            pl.BlockSpec(memory_space=pl.ANY),
            pl.BlockSpec(memory_space=pl.ANY),
        ],
        out_specs=pl.BlockSpec(memory_space=pl.ANY),
        scratch_shapes=[
            pltpu.VMEM((2, m_blk, k_shard), jnp.float32),
            pltpu.VMEM((N_DEV, m_blk, k_shard), jnp.bfloat16),
            pltpu.VMEM((N_DEV, m_blk, k_shard), jnp.bfloat16),
            pltpu.VMEM((2, k_blk, n), jnp.float32),
            pltpu.VMEM((m_blk, n), jnp.float32),
            pltpu.VMEM((8, 128), jnp.float32),
            pltpu.VMEM((N_DEV, 8, 128), jnp.float32),
            pltpu.SemaphoreType.DMA((2,)),
            pltpu.SemaphoreType.DMA((2,)),
            pltpu.SemaphoreType.DMA((3, 4)),
            pltpu.SemaphoreType.DMA((N_DEV, 4)),
            pltpu.SemaphoreType.DMA((3,)),
            pltpu.SemaphoreType.DMA((4,)),
            pltpu.SemaphoreType.DMA((2,)),
        ],
        compiler_params=pltpu.CompilerParams(
            collective_id=0,
            vmem_limit_bytes=63 * 1024 * 1024,
        ),
    )(x, w_mat)
